# baseline (speedup 1.0000x reference)
import functools

import jax
import jax.numpy as jnp
from jax import lax
from jax.experimental import pallas as pl
from jax.experimental.pallas import tpu as pltpu
from jax.experimental.pallas import tpu_sc as plsc

N = 10000
D = 128
H = 3
E = 320000
E_PAD = 327680
VOCAB = 22754
VOCAB_PAD = 22784
DEN_PAD = 40064
NEG_SLOPE = 0.2
BLK = 1000
_HIGH = lax.Precision.HIGHEST

NC = 2
NS = 16
NW = NC * NS

K1_PER_TILE = E_PAD // NW
K1_CHUNK = 512
K1_STEPS = K1_PER_TILE // K1_CHUNK
K2_PER_TILE = E_PAD // NS
K2_CHUNK = 256
K2_STEPS = K2_PER_TILE // K2_CHUNK
NROW = N // NS


def _pre_body(x_ref, wl_ref, bl_ref, wh_ref, asrc_ref, adst_ref,
              h0_ref, s_ref):
    xb = x_ref[...]
    h0 = jnp.dot(xb, wl_ref[...], preferred_element_type=jnp.float32,
                 precision=_HIGH) + bl_ref[...]
    wh = wh_ref[...]
    csrc = lax.dot_general(wh, asrc_ref[...], (((2,), (1,)), ((0,), (0,))),
                           preferred_element_type=jnp.float32,
                           precision=_HIGH)
    cdst = lax.dot_general(wh, adst_ref[...], (((2,), (1,)), ((0,), (0,))),
                           preferred_element_type=jnp.float32,
                           precision=_HIGH)
    z1 = jnp.zeros((1, D), jnp.float32)
    z8 = jnp.zeros((8, D), jnp.float32)
    c16 = jnp.concatenate([csrc, z1, cdst, z1, z8], axis=0)
    s_ref[...] = lax.dot_general(h0, c16, (((1,), (1,)), ((), ())),
                                 preferred_element_type=jnp.float32,
                                 precision=_HIGH)
    for q in range(4):
        h0_ref[q] = h0[:, q * 32:(q + 1) * 32]


def _pre(x, w_lin, b_lin, w_heads, att_src, att_dst):
    return pl.pallas_call(
        _pre_body,
        grid=(N // BLK,),
        in_specs=[
            pl.BlockSpec((BLK, D), lambda i: (i, 0)),
            pl.BlockSpec((D, D), lambda i: (0, 0)),
            pl.BlockSpec((1, D), lambda i: (0, 0)),
            pl.BlockSpec((H, D, D), lambda i: (0, 0, 0)),
            pl.BlockSpec((H, D), lambda i: (0, 0)),
            pl.BlockSpec((H, D), lambda i: (0, 0)),
        ],
        out_specs=[
            pl.BlockSpec((4, BLK, 32), lambda i: (0, i, 0)),
            pl.BlockSpec((BLK, 16), lambda i: (i, 0)),
        ],
        out_shape=[
            jax.ShapeDtypeStruct((4, N, 32), jnp.float32),
            jax.ShapeDtypeStruct((N, 16), jnp.float32),
        ],
    )(x, w_lin, b_lin.reshape(1, D), w_heads, att_src, att_dst)


def _sc_edges_body(src_hbm, dst_hbm, ew_hbm, ddi_hbm, s16_hbm, emb_hbm,
                   u_hbm, den_hbm,
                   srcb, dstb, ewb, ddib, srow, drow, u0b, u1b, u2b,
                   denbuf, embbuf, insem, gsem, usem):
    cid = lax.axis_index("c")
    sid = lax.axis_index("s")
    wid = sid * NC + cid
    iota = lax.iota(jnp.int32, 16)
    ubufs = (u0b, u1b, u2b)

    pltpu.sync_copy(emb_hbm, embbuf)

    def zero_body(i, carry):
        denbuf[pl.ds(i * 16, 16)] = jnp.zeros((16,), jnp.float32)
        return carry
    lax.fori_loop(0, DEN_PAD // 16, zero_body, 0)

    tile_base = wid * K1_PER_TILE

    def in_descs(k, par):
        kc = lax.min(k, K1_STEPS - 1)
        base = lax.min(tile_base + kc * K1_CHUNK, E - K1_CHUNK)
        return (pltpu.make_async_copy(src_hbm.at[pl.ds(base, K1_CHUNK)],
                                      srcb.at[par], insem),
                pltpu.make_async_copy(dst_hbm.at[pl.ds(base, K1_CHUNK)],
                                      dstb.at[par], insem),
                pltpu.make_async_copy(ew_hbm.at[pl.ds(base, K1_CHUNK)],
                                      ewb.at[par], insem),
                pltpu.make_async_copy(ddi_hbm.at[pl.ds(base, K1_CHUNK)],
                                      ddib.at[par], insem))

    def issue_in(k, par):
        for d in in_descs(k, par):
            d.start()

    def wait_in(k, par):
        for d in in_descs(k, par):
            d.wait()

    def gather_descs(par):
        out = []
        for r in range(K1_CHUNK // 128):
            out.append(pltpu.make_async_copy(
                s16_hbm.at[srcb.at[par, pl.ds(r * 128, 128)]],
                srow.at[par, pl.ds(r * 128, 128)], gsem))
            out.append(pltpu.make_async_copy(
                s16_hbm.at[dstb.at[par, pl.ds(r * 128, 128)]],
                drow.at[par, pl.ds(r * 128, 128)], gsem))
        return out

    def uout_descs(k, par):
        kc = lax.min(k, K1_STEPS - 1)
        base = lax.min(tile_base + kc * K1_CHUNK, E - K1_CHUNK)
        return tuple(
            pltpu.make_async_copy(ubufs[hd].at[par],
                                  u_hbm.at[hd, pl.ds(base, K1_CHUNK)], usem)
            for hd in range(H))

    for d in in_descs(0, 0):
        d.start()
    for d in in_descs(0, 0):
        d.wait()
    for d in gather_descs(0):
        d.start()

    def chunk_body(k, carry):
        b = lax.rem(k, 2)
        nb = 1 - b
        base = tile_base + k * K1_CHUNK
        issue_in(k + 1, nb)
        for d in gather_descs(b):
            d.wait()

        @pl.when((k != 0) & (tile_base + (k - 1) * K1_CHUNK < E))
        def _():
            for d in uout_descs(k - 1, nb):
                d.wait()

        bv = jnp.full((16,), b, jnp.int32)
        for j in range(K1_CHUNK // 16):
            dst_v = dstb[b, pl.ds(j * 16, 16)]
            ewi_v = ewb[b, pl.ds(j * 16, 16)]
            ddi_v = ddib[b, pl.ds(j * 16, 16)]
            ew_v = plsc.load_gather(embbuf, [ewi_v]) - ddi_v
            rowv = iota + (j * 16)
            valid = (iota + (base + j * 16)) < E
            den_base = dst_v * 4
            for hd in range(H):
                s1 = plsc.load_gather(srow, [bv, rowv,
                                             jnp.full((16,), hd, jnp.int32)])
                s2 = plsc.load_gather(drow, [bv, rowv,
                                             jnp.full((16,), hd + 4,
                                                      jnp.int32)])
                a = s1 + s2
                p = jnp.exp(jnp.maximum(a, NEG_SLOPE * a))
                p = jnp.where(valid, p, 0.0)
                plsc.addupdate_scatter(denbuf, [den_base + hd], p)
                ubufs[hd][b, pl.ds(j * 16, 16)] = ew_v * p
        @pl.when(tile_base + k * K1_CHUNK < E)
        def _():
            for d in uout_descs(k, b):
                d.start()
        wait_in(k + 1, nb)
        for d in gather_descs(nb):
            d.start()
        return carry

    lax.fori_loop(0, K1_STEPS, chunk_body, 0)

    @pl.when(tile_base + (K1_STEPS - 1) * K1_CHUNK < E)
    def _():
        for d in uout_descs(K1_STEPS - 1, lax.rem(K1_STEPS - 1, 2)):
            d.wait()
    for d in gather_descs(lax.rem(K1_STEPS, 2)):
        d.wait()
    pltpu.sync_copy(denbuf.at[pl.ds(0, N * 4)], den_hbm.at[wid])


def _sc_edges(srcp, dstp, ewp, ddip, s16, emb):
    mesh = plsc.VectorSubcoreMesh(core_axis_name="c", subcore_axis_name="s")
    f = pl.kernel(
        _sc_edges_body,
        out_type=[
            jax.ShapeDtypeStruct((H, E_PAD), jnp.float32),
            jax.ShapeDtypeStruct((NW, N * 4), jnp.float32),
        ],
        mesh=mesh,
        compiler_params=pltpu.CompilerParams(needs_layout_passes=False,
                                             use_tc_tiling_on_sc=False),
        scratch_types=[
            pltpu.VMEM((2, K1_CHUNK), jnp.int32),
            pltpu.VMEM((2, K1_CHUNK), jnp.int32),
            pltpu.VMEM((2, K1_CHUNK), jnp.int32),
            pltpu.VMEM((2, K1_CHUNK), jnp.float32),
            pltpu.VMEM((2, K1_CHUNK, 16), jnp.float32),
            pltpu.VMEM((2, K1_CHUNK, 16), jnp.float32),
            pltpu.VMEM((2, K1_CHUNK), jnp.float32),
            pltpu.VMEM((2, K1_CHUNK), jnp.float32),
            pltpu.VMEM((2, K1_CHUNK), jnp.float32),
            pltpu.VMEM((DEN_PAD,), jnp.float32),
            pltpu.VMEM((VOCAB_PAD,), jnp.float32),
            pltpu.SemaphoreType.DMA,
            pltpu.SemaphoreType.DMA,
            pltpu.SemaphoreType.DMA,
        ],
    )
    return f(srcp, dstp, ewp, ddip, s16, emb)


def _sc_agg_body(src_hbm, dst2_hbm, u_hbm, h0q_hbm, agg_hbm,
                 srcb, dstb2, idxb, ubuf, rbuf, w96,
                 zbuf, acc96, insem, gsem, ssem):
    cid = lax.axis_index("c")
    sid = lax.axis_index("s")
    NG = K2_CHUNK // 128

    def zero_body16(i, carry):
        r = i // 6
        g = i % 6
        zbuf[r, pl.ds(g * 16, 16)] = jnp.zeros((16,), jnp.float32)
        return carry
    lax.fori_loop(0, 125 * 6, zero_body16, 0)

    tile_base = sid * K2_PER_TILE

    def issue_in(k, par):
        kc = lax.min(k, K2_STEPS - 1)
        base = lax.min(tile_base + kc * K2_CHUNK, E - K2_CHUNK)
        row0 = base // 128
        d1 = pltpu.async_copy(src_hbm.at[pl.ds(base, K2_CHUNK)],
                              srcb.at[par], insem)
        d2 = pltpu.async_copy(dst2_hbm.at[pl.ds(row0, NG)],
                              dstb2.at[par], insem)
        d3 = pltpu.async_copy(
            u_hbm.at[0, pl.ds(base, K2_CHUNK)],
            ubuf.at[pl.ds(par * 3 * K2_CHUNK, K2_CHUNK)], insem)
        d4 = pltpu.async_copy(
            u_hbm.at[1, pl.ds(base, K2_CHUNK)],
            ubuf.at[pl.ds(par * 3 * K2_CHUNK + K2_CHUNK, K2_CHUNK)], insem)
        d5 = pltpu.async_copy(
            u_hbm.at[2, pl.ds(base, K2_CHUNK)],
            ubuf.at[pl.ds(par * 3 * K2_CHUNK + 2 * K2_CHUNK, K2_CHUNK)],
            insem)
        return d1, d2, d3, d4, d5

    def wait_in(k, par):
        for d in issue_wait_in_descs(k, par):
            d.wait()

    def issue_wait_in_descs(k, par):
        kc = lax.min(k, K2_STEPS - 1)
        base = lax.min(tile_base + kc * K2_CHUNK, E - K2_CHUNK)
        row0 = base // 128
        return (pltpu.make_async_copy(src_hbm.at[pl.ds(base, K2_CHUNK)],
                                      srcb.at[par], insem),
                pltpu.make_async_copy(dst2_hbm.at[pl.ds(row0, NG)],
                                      dstb2.at[par], insem),
                pltpu.make_async_copy(
                    u_hbm.at[0, pl.ds(base, K2_CHUNK)],
                    ubuf.at[pl.ds(par * 3 * K2_CHUNK, K2_CHUNK)], insem),
                pltpu.make_async_copy(
                    u_hbm.at[1, pl.ds(base, K2_CHUNK)],
                    ubuf.at[pl.ds(par * 3 * K2_CHUNK + K2_CHUNK, K2_CHUNK)],
                    insem),
                pltpu.make_async_copy(
                    u_hbm.at[2, pl.ds(base, K2_CHUNK)],
                    ubuf.at[pl.ds(par * 3 * K2_CHUNK + 2 * K2_CHUNK,
                                  K2_CHUNK)], insem))

    def compute_idx_issue_gather(par, qN):
        for g in range(K2_CHUNK // 16):
            r = g // 8
            t0 = (g % 8) * 16
            idxb[par, r, pl.ds(t0, 16)] = srcb[par, pl.ds(g * 16, 16)] + qN
        for r in range(NG):
            pltpu.async_copy(h0q_hbm.at[idxb.at[par, r]],
                             rbuf.at[par, pl.ds(r * 128, 128)], gsem)

    def wait_gather(par):
        for r in range(NG):
            pltpu.make_async_copy(
                h0q_hbm.at[idxb.at[par, r]],
                rbuf.at[par, pl.ds(r * 128, 128)], gsem).wait()

    def wait_scatter_all():
        for _ in range(NG):
            pltpu.make_async_copy(agg_hbm.at[0, pl.ds(0, 128)],
                                  w96.at[pl.ds(0, 128)], ssem).wait()

    for q in range(2):
        qq = cid * 2 + q
        qN = jnp.full((16,), qq * N, jnp.int32)

        for r5 in range(5):
            pltpu.sync_copy(zbuf,
                            acc96.at[pl.ds(sid * NROW + r5 * 125, 125)])
        plsc.subcore_barrier()

        for d in issue_in(0, 0):
            d.wait()
        compute_idx_issue_gather(0, qN)

        def chunk_body(k, carry):
            b = lax.rem(k, 2)
            nb = 1 - b
            issue_in(k + 1, nb)
            wait_gather(b)

            @pl.when(k != 0)
            def _():
                wait_scatter_all()

            ub0 = b * (3 * K2_CHUNK)
            kbase = tile_base + k * K2_CHUNK

            def edge_body(e, carry2):
                i0 = jnp.full((16,), ub0 + e, jnp.int32)
                ok = (jnp.full((16,), kbase + e, jnp.int32)
                      < jnp.full((16,), E, jnp.int32))
                u0 = jnp.where(ok, plsc.load_gather(ubuf, [i0]), 0.0)
                u1 = jnp.where(ok, plsc.load_gather(ubuf, [i0 + K2_CHUNK]),
                               0.0)
                u2 = jnp.where(ok, plsc.load_gather(ubuf,
                                                    [i0 + 2 * K2_CHUNK]),
                               0.0)
                for g4 in range(2):
                    r_v = rbuf[b, e, pl.ds(g4 * 16, 16)]
                    w96[e, pl.ds(g4 * 16, 16)] = r_v * u0
                    w96[e, pl.ds(32 + g4 * 16, 16)] = r_v * u1
                    w96[e, pl.ds(64 + g4 * 16, 16)] = r_v * u2
                return carry2

            for r in range(NG):
                lax.fori_loop(r * 128, (r + 1) * 128, edge_body, 0,
                              unroll=8)
                pltpu.async_copy(w96.at[pl.ds(r * 128, 128)],
                                 acc96.at[dstb2.at[b, r]], ssem, add=True)

            wait_in(k + 1, nb)
            compute_idx_issue_gather(nb, qN)
            return carry

        lax.fori_loop(0, K2_STEPS, chunk_body, 0)
        wait_scatter_all()
        wait_gather(lax.rem(K2_STEPS, 2))
        plsc.subcore_barrier()
        pltpu.sync_copy(acc96.at[pl.ds(sid * NROW, NROW)],
                        agg_hbm.at[qq, pl.ds(sid * NROW, NROW)])
        plsc.subcore_barrier()


def _sc_agg(srcp, dst2, u, h0quar):
    mesh = plsc.VectorSubcoreMesh(core_axis_name="c", subcore_axis_name="s")
    f = pl.kernel(
        _sc_agg_body,
        out_type=jax.ShapeDtypeStruct((4, N, 96), jnp.float32),
        mesh=mesh,
        compiler_params=pltpu.CompilerParams(needs_layout_passes=False,
                                             use_tc_tiling_on_sc=False),
        scratch_types=[
            pltpu.VMEM((2, K2_CHUNK), jnp.int32),
            pltpu.VMEM((2, K2_CHUNK // 128, 128), jnp.int32),
            pltpu.VMEM((2, K2_CHUNK // 128, 128), jnp.int32),
            pltpu.VMEM((2 * H * K2_CHUNK,), jnp.float32),
            pltpu.VMEM((2, K2_CHUNK, 32), jnp.float32),
            pltpu.VMEM((K2_CHUNK, 96), jnp.float32),
            pltpu.VMEM((125, 96), jnp.float32),
            pltpu.VMEM_SHARED((N, 96), jnp.float32),
            pltpu.SemaphoreType.DMA,
            pltpu.SemaphoreType.DMA,
            pltpu.SemaphoreType.DMA,
        ],
    )
    return f(srcp, dst2, u, h0quar)


def _post_body(agg_ref, den_ref, wh_ref, bias_ref, out_ref):
    den = jnp.sum(den_ref[...], axis=0)
    acc = jnp.zeros((BLK, D), jnp.float32)
    for hd in range(H):
        inv = 1.0 / (den[:, hd][:, None] + 1e-16)
        for qq in range(4):
            acc = acc + jnp.dot(agg_ref[qq][:, hd * 32:(hd + 1) * 32] * inv,
                                wh_ref[hd, qq * 32:(qq + 1) * 32, :],
                                preferred_element_type=jnp.float32,
                                precision=_HIGH)
    out_ref[...] = acc * (1.0 / H) + jnp.mean(bias_ref[...], axis=0)


def _post(agg, den32, w_heads, bias_heads):
    return pl.pallas_call(
        _post_body,
        grid=(N // BLK,),
        in_specs=[
            pl.BlockSpec((4, BLK, 96), lambda i: (0, i, 0)),
            pl.BlockSpec((NW, BLK, 4), lambda i: (0, i, 0)),
            pl.BlockSpec((H, D, D), lambda i: (0, 0, 0)),
            pl.BlockSpec((H, D), lambda i: (0, 0)),
        ],
        out_specs=pl.BlockSpec((BLK, D), lambda i: (i, 0)),
        out_shape=jax.ShapeDtypeStruct((N, D), jnp.float32),
    )(agg, den32, w_heads, bias_heads)


def kernel(x, edge_index, edge_weight, ddi_weight, W_lin, b_lin, edge_emb,
           W_heads, att_src, att_dst, bias_heads):
    h0pair, s16 = _pre(x, W_lin, b_lin, W_heads, att_src, att_dst)
    h0quar = h0pair.reshape(4 * N, 32)

    srcp = edge_index[0]
    dstp = edge_index[1]
    embp = jnp.concatenate([edge_emb[:, 0],
                            jnp.zeros((VOCAB_PAD - VOCAB,), jnp.float32)])

    u, den = _sc_edges(srcp, dstp, edge_weight, ddi_weight, s16, embp)
    agg = _sc_agg(srcp, dstp.reshape(-1, 128), u, h0quar)
    return _post(agg, den.reshape(NW, N, 4), W_heads, bias_heads)


# speedup vs baseline: 1.0145x; 1.0145x over previous
import functools

import jax
import jax.numpy as jnp
from jax import lax
from jax.experimental import pallas as pl
from jax.experimental.pallas import tpu as pltpu
from jax.experimental.pallas import tpu_sc as plsc

N = 10000
D = 128
H = 3
E = 320000
E_PAD = 327680
VOCAB = 22754
VOCAB_PAD = 22784
DEN_PAD = 40064
NEG_SLOPE = 0.2
BLK = 1000
_HIGH = lax.Precision.HIGHEST

NC = 2
NS = 16
NW = NC * NS

K1_PER_TILE = E_PAD // NW
K1_CHUNK = 512
K1_STEPS = K1_PER_TILE // K1_CHUNK
K2_PER_TILE = E_PAD // NS
K2_CHUNK = 256
K2_STEPS = K2_PER_TILE // K2_CHUNK
NROW = N // NS


def _pre_body(x_ref, wl_ref, bl_ref, wh_ref, asrc_ref, adst_ref,
              h0_ref, s_ref):
    xb = x_ref[...]
    h0 = jnp.dot(xb, wl_ref[...], preferred_element_type=jnp.float32,
                 precision=_HIGH) + bl_ref[...]
    wh = wh_ref[...]
    csrc = lax.dot_general(wh, asrc_ref[...], (((2,), (1,)), ((0,), (0,))),
                           preferred_element_type=jnp.float32,
                           precision=_HIGH)
    cdst = lax.dot_general(wh, adst_ref[...], (((2,), (1,)), ((0,), (0,))),
                           preferred_element_type=jnp.float32,
                           precision=_HIGH)
    z1 = jnp.zeros((1, D), jnp.float32)
    z8 = jnp.zeros((8, D), jnp.float32)
    c16 = jnp.concatenate([csrc, z1, cdst, z1, z8], axis=0)
    s_ref[...] = lax.dot_general(h0, c16, (((1,), (1,)), ((), ())),
                                 preferred_element_type=jnp.float32,
                                 precision=_HIGH)
    for q in range(4):
        h0_ref[q] = h0[:, q * 32:(q + 1) * 32]


def _pre(x, w_lin, b_lin, w_heads, att_src, att_dst):
    return pl.pallas_call(
        _pre_body,
        grid=(N // BLK,),
        in_specs=[
            pl.BlockSpec((BLK, D), lambda i: (i, 0)),
            pl.BlockSpec((D, D), lambda i: (0, 0)),
            pl.BlockSpec((1, D), lambda i: (0, 0)),
            pl.BlockSpec((H, D, D), lambda i: (0, 0, 0)),
            pl.BlockSpec((H, D), lambda i: (0, 0)),
            pl.BlockSpec((H, D), lambda i: (0, 0)),
        ],
        out_specs=[
            pl.BlockSpec((4, BLK, 32), lambda i: (0, i, 0)),
            pl.BlockSpec((BLK, 16), lambda i: (i, 0)),
        ],
        out_shape=[
            jax.ShapeDtypeStruct((4, N, 32), jnp.float32),
            jax.ShapeDtypeStruct((N, 16), jnp.float32),
        ],
    )(x, w_lin, b_lin.reshape(1, D), w_heads, att_src, att_dst)


def _sc_edges_body(src_hbm, dst_hbm, ew_hbm, ddi_hbm, s16_hbm, emb_hbm,
                   u_hbm, den_hbm,
                   srcb, dstb, ewb, ddib, srow, drow, u0b, u1b, u2b,
                   denbuf, embbuf, insem, gsem, usem):
    cid = lax.axis_index("c")
    sid = lax.axis_index("s")
    wid = sid * NC + cid
    iota = lax.iota(jnp.int32, 16)
    ubufs = (u0b, u1b, u2b)

    pltpu.sync_copy(emb_hbm, embbuf)

    def zero_body(i, carry):
        denbuf[pl.ds(i * 16, 16)] = jnp.zeros((16,), jnp.float32)
        return carry
    lax.fori_loop(0, DEN_PAD // 16, zero_body, 0)

    tile_base = wid * K1_PER_TILE

    def in_descs(k, par):
        kc = lax.min(k, K1_STEPS - 1)
        base = tile_base + kc * K1_CHUNK
        return (pltpu.make_async_copy(src_hbm.at[pl.ds(base, K1_CHUNK)],
                                      srcb.at[par], insem),
                pltpu.make_async_copy(dst_hbm.at[pl.ds(base, K1_CHUNK)],
                                      dstb.at[par], insem),
                pltpu.make_async_copy(ew_hbm.at[pl.ds(base, K1_CHUNK)],
                                      ewb.at[par], insem),
                pltpu.make_async_copy(ddi_hbm.at[pl.ds(base, K1_CHUNK)],
                                      ddib.at[par], insem))

    def issue_in(k, par):
        for d in in_descs(k, par):
            d.start()

    def wait_in(k, par):
        for d in in_descs(k, par):
            d.wait()

    def gather_descs(par):
        out = []
        for r in range(K1_CHUNK // 128):
            out.append(pltpu.make_async_copy(
                s16_hbm.at[srcb.at[par, pl.ds(r * 128, 128)]],
                srow.at[par, pl.ds(r * 128, 128)], gsem))
            out.append(pltpu.make_async_copy(
                s16_hbm.at[dstb.at[par, pl.ds(r * 128, 128)]],
                drow.at[par, pl.ds(r * 128, 128)], gsem))
        return out

    def uout_descs(k, par):
        kc = lax.min(k, K1_STEPS - 1)
        base = tile_base + kc * K1_CHUNK
        return tuple(
            pltpu.make_async_copy(ubufs[hd].at[par],
                                  u_hbm.at[hd, pl.ds(base, K1_CHUNK)], usem)
            for hd in range(H))

    for d in in_descs(0, 0):
        d.start()
    for d in in_descs(0, 0):
        d.wait()
    for d in gather_descs(0):
        d.start()

    def chunk_body(k, carry):
        b = lax.rem(k, 2)
        nb = 1 - b
        base = tile_base + k * K1_CHUNK
        issue_in(k + 1, nb)
        for d in gather_descs(b):
            d.wait()

        @pl.when(k != 0)
        def _():
            for d in uout_descs(k - 1, nb):
                d.wait()

        bv = jnp.full((16,), b, jnp.int32)
        for j in range(K1_CHUNK // 16):
            dst_v = dstb[b, pl.ds(j * 16, 16)]
            ewi_v = ewb[b, pl.ds(j * 16, 16)]
            ddi_v = ddib[b, pl.ds(j * 16, 16)]
            ew_v = plsc.load_gather(embbuf, [ewi_v]) - ddi_v
            rowv = iota + (j * 16)
            valid = (iota + (base + j * 16)) < E
            den_base = dst_v * 4
            for hd in range(H):
                s1 = plsc.load_gather(srow, [bv, rowv,
                                             jnp.full((16,), hd, jnp.int32)])
                s2 = plsc.load_gather(drow, [bv, rowv,
                                             jnp.full((16,), hd + 4,
                                                      jnp.int32)])
                a = s1 + s2
                p = jnp.exp(jnp.maximum(a, NEG_SLOPE * a))
                p = jnp.where(valid, p, 0.0)
                plsc.addupdate_scatter(denbuf, [den_base + hd], p)
                ubufs[hd][b, pl.ds(j * 16, 16)] = ew_v * p
        for d in uout_descs(k, b):
            d.start()
        wait_in(k + 1, nb)
        for d in gather_descs(nb):
            d.start()
        return carry

    lax.fori_loop(0, K1_STEPS, chunk_body, 0)
    for d in uout_descs(K1_STEPS - 1, lax.rem(K1_STEPS - 1, 2)):
        d.wait()
    for d in gather_descs(lax.rem(K1_STEPS, 2)):
        d.wait()
    pltpu.sync_copy(denbuf.at[pl.ds(0, N * 4)], den_hbm.at[wid])


def _sc_edges(srcp, dstp, ewp, ddip, s16, emb):
    mesh = plsc.VectorSubcoreMesh(core_axis_name="c", subcore_axis_name="s")
    f = pl.kernel(
        _sc_edges_body,
        out_type=[
            jax.ShapeDtypeStruct((H, E_PAD), jnp.float32),
            jax.ShapeDtypeStruct((NW, N * 4), jnp.float32),
        ],
        mesh=mesh,
        compiler_params=pltpu.CompilerParams(needs_layout_passes=False,
                                             use_tc_tiling_on_sc=False),
        scratch_types=[
            pltpu.VMEM((2, K1_CHUNK), jnp.int32),
            pltpu.VMEM((2, K1_CHUNK), jnp.int32),
            pltpu.VMEM((2, K1_CHUNK), jnp.int32),
            pltpu.VMEM((2, K1_CHUNK), jnp.float32),
            pltpu.VMEM((2, K1_CHUNK, 16), jnp.float32),
            pltpu.VMEM((2, K1_CHUNK, 16), jnp.float32),
            pltpu.VMEM((2, K1_CHUNK), jnp.float32),
            pltpu.VMEM((2, K1_CHUNK), jnp.float32),
            pltpu.VMEM((2, K1_CHUNK), jnp.float32),
            pltpu.VMEM((DEN_PAD,), jnp.float32),
            pltpu.VMEM((VOCAB_PAD,), jnp.float32),
            pltpu.SemaphoreType.DMA,
            pltpu.SemaphoreType.DMA,
            pltpu.SemaphoreType.DMA,
        ],
    )
    return f(srcp, dstp, ewp, ddip, s16, emb)


def _sc_agg_body(src_hbm, dst2_hbm, u_hbm, h0q_hbm, agg_hbm,
                 srcb, dstb2, idxb, ubuf, rbuf, w96,
                 zbuf, acc96, insem, gsem, ssem):
    cid = lax.axis_index("c")
    sid = lax.axis_index("s")
    NG = K2_CHUNK // 128

    def zero_body16(i, carry):
        r = i // 6
        g = i % 6
        zbuf[r, pl.ds(g * 16, 16)] = jnp.zeros((16,), jnp.float32)
        return carry
    lax.fori_loop(0, 125 * 6, zero_body16, 0)

    tile_base = sid * K2_PER_TILE

    def issue_in(k, par):
        kc = lax.min(k, K2_STEPS - 1)
        base = tile_base + kc * K2_CHUNK
        row0 = base // 128
        d1 = pltpu.async_copy(src_hbm.at[pl.ds(base, K2_CHUNK)],
                              srcb.at[par], insem)
        d2 = pltpu.async_copy(dst2_hbm.at[pl.ds(row0, NG)],
                              dstb2.at[par], insem)
        d3 = pltpu.async_copy(
            u_hbm.at[0, pl.ds(base, K2_CHUNK)],
            ubuf.at[pl.ds(par * 3 * K2_CHUNK, K2_CHUNK)], insem)
        d4 = pltpu.async_copy(
            u_hbm.at[1, pl.ds(base, K2_CHUNK)],
            ubuf.at[pl.ds(par * 3 * K2_CHUNK + K2_CHUNK, K2_CHUNK)], insem)
        d5 = pltpu.async_copy(
            u_hbm.at[2, pl.ds(base, K2_CHUNK)],
            ubuf.at[pl.ds(par * 3 * K2_CHUNK + 2 * K2_CHUNK, K2_CHUNK)],
            insem)
        return d1, d2, d3, d4, d5

    def wait_in(k, par):
        for d in issue_wait_in_descs(k, par):
            d.wait()

    def issue_wait_in_descs(k, par):
        kc = lax.min(k, K2_STEPS - 1)
        base = tile_base + kc * K2_CHUNK
        row0 = base // 128
        return (pltpu.make_async_copy(src_hbm.at[pl.ds(base, K2_CHUNK)],
                                      srcb.at[par], insem),
                pltpu.make_async_copy(dst2_hbm.at[pl.ds(row0, NG)],
                                      dstb2.at[par], insem),
                pltpu.make_async_copy(
                    u_hbm.at[0, pl.ds(base, K2_CHUNK)],
                    ubuf.at[pl.ds(par * 3 * K2_CHUNK, K2_CHUNK)], insem),
                pltpu.make_async_copy(
                    u_hbm.at[1, pl.ds(base, K2_CHUNK)],
                    ubuf.at[pl.ds(par * 3 * K2_CHUNK + K2_CHUNK, K2_CHUNK)],
                    insem),
                pltpu.make_async_copy(
                    u_hbm.at[2, pl.ds(base, K2_CHUNK)],
                    ubuf.at[pl.ds(par * 3 * K2_CHUNK + 2 * K2_CHUNK,
                                  K2_CHUNK)], insem))

    def compute_idx_issue_gather(par, qN):
        for g in range(K2_CHUNK // 16):
            r = g // 8
            t0 = (g % 8) * 16
            idxb[par, r, pl.ds(t0, 16)] = srcb[par, pl.ds(g * 16, 16)] + qN
        for r in range(NG):
            pltpu.async_copy(h0q_hbm.at[idxb.at[par, r]],
                             rbuf.at[par, pl.ds(r * 128, 128)], gsem)

    def wait_gather(par):
        for r in range(NG):
            pltpu.make_async_copy(
                h0q_hbm.at[idxb.at[par, r]],
                rbuf.at[par, pl.ds(r * 128, 128)], gsem).wait()

    def wait_scatter_all():
        for _ in range(NG):
            pltpu.make_async_copy(agg_hbm.at[0, pl.ds(0, 128)],
                                  w96.at[pl.ds(0, 128)], ssem).wait()

    for q in range(2):
        qq = cid * 2 + q
        qN = jnp.full((16,), qq * N, jnp.int32)

        for r5 in range(5):
            pltpu.sync_copy(zbuf,
                            acc96.at[pl.ds(sid * NROW + r5 * 125, 125)])
        plsc.subcore_barrier()

        for d in issue_in(0, 0):
            d.wait()
        compute_idx_issue_gather(0, qN)

        def chunk_body(k, carry):
            b = lax.rem(k, 2)
            nb = 1 - b
            issue_in(k + 1, nb)
            wait_gather(b)

            @pl.when(k != 0)
            def _():
                wait_scatter_all()

            ub0 = b * (3 * K2_CHUNK)

            def edge_body(e, carry2):
                i0 = jnp.full((16,), ub0 + e, jnp.int32)
                u0 = plsc.load_gather(ubuf, [i0])
                u1 = plsc.load_gather(ubuf, [i0 + K2_CHUNK])
                u2 = plsc.load_gather(ubuf, [i0 + 2 * K2_CHUNK])
                for g4 in range(2):
                    r_v = rbuf[b, e, pl.ds(g4 * 16, 16)]
                    w96[e, pl.ds(g4 * 16, 16)] = r_v * u0
                    w96[e, pl.ds(32 + g4 * 16, 16)] = r_v * u1
                    w96[e, pl.ds(64 + g4 * 16, 16)] = r_v * u2
                return carry2

            for r in range(NG):
                lax.fori_loop(r * 128, (r + 1) * 128, edge_body, 0,
                              unroll=8)
                pltpu.async_copy(w96.at[pl.ds(r * 128, 128)],
                                 acc96.at[dstb2.at[b, r]], ssem, add=True)

            wait_in(k + 1, nb)
            compute_idx_issue_gather(nb, qN)
            return carry

        lax.fori_loop(0, K2_STEPS, chunk_body, 0)
        wait_scatter_all()
        wait_gather(lax.rem(K2_STEPS, 2))
        plsc.subcore_barrier()
        pltpu.sync_copy(acc96.at[pl.ds(sid * NROW, NROW)],
                        agg_hbm.at[qq, pl.ds(sid * NROW, NROW)])
        plsc.subcore_barrier()


def _sc_agg(srcp, dst2, u, h0quar):
    mesh = plsc.VectorSubcoreMesh(core_axis_name="c", subcore_axis_name="s")
    f = pl.kernel(
        _sc_agg_body,
        out_type=jax.ShapeDtypeStruct((4, N, 96), jnp.float32),
        mesh=mesh,
        compiler_params=pltpu.CompilerParams(needs_layout_passes=False,
                                             use_tc_tiling_on_sc=False),
        scratch_types=[
            pltpu.VMEM((2, K2_CHUNK), jnp.int32),
            pltpu.VMEM((2, K2_CHUNK // 128, 128), jnp.int32),
            pltpu.VMEM((2, K2_CHUNK // 128, 128), jnp.int32),
            pltpu.VMEM((2 * H * K2_CHUNK,), jnp.float32),
            pltpu.VMEM((2, K2_CHUNK, 32), jnp.float32),
            pltpu.VMEM((K2_CHUNK, 96), jnp.float32),
            pltpu.VMEM((125, 96), jnp.float32),
            pltpu.VMEM_SHARED((N, 96), jnp.float32),
            pltpu.SemaphoreType.DMA,
            pltpu.SemaphoreType.DMA,
            pltpu.SemaphoreType.DMA,
        ],
    )
    return f(srcp, dst2, u, h0quar)


def _post_body(agg_ref, den_ref, wh_ref, bias_ref, out_ref):
    den = jnp.sum(den_ref[...], axis=0)
    acc = jnp.zeros((BLK, D), jnp.float32)
    for hd in range(H):
        inv = 1.0 / (den[:, hd][:, None] + 1e-16)
        for qq in range(4):
            acc = acc + jnp.dot(agg_ref[qq][:, hd * 32:(hd + 1) * 32] * inv,
                                wh_ref[hd, qq * 32:(qq + 1) * 32, :],
                                preferred_element_type=jnp.float32,
                                precision=_HIGH)
    out_ref[...] = acc * (1.0 / H) + jnp.mean(bias_ref[...], axis=0)


def _post(agg, den32, w_heads, bias_heads):
    return pl.pallas_call(
        _post_body,
        grid=(N // BLK,),
        in_specs=[
            pl.BlockSpec((4, BLK, 96), lambda i: (0, i, 0)),
            pl.BlockSpec((NW, BLK, 4), lambda i: (0, i, 0)),
            pl.BlockSpec((H, D, D), lambda i: (0, 0, 0)),
            pl.BlockSpec((H, D), lambda i: (0, 0)),
        ],
        out_specs=pl.BlockSpec((BLK, D), lambda i: (i, 0)),
        out_shape=jax.ShapeDtypeStruct((N, D), jnp.float32),
    )(agg, den32, w_heads, bias_heads)


def kernel(x, edge_index, edge_weight, ddi_weight, W_lin, b_lin, edge_emb,
           W_heads, att_src, att_dst, bias_heads):
    h0pair, s16 = _pre(x, W_lin, b_lin, W_heads, att_src, att_dst)
    h0quar = h0pair.reshape(4 * N, 32)

    pad = E_PAD - E
    spread = (jnp.arange(pad, dtype=jnp.int32) * 13) % N
    srcp = jnp.concatenate([edge_index[0], spread])
    dstp = jnp.concatenate([edge_index[1], spread])
    ewp = jnp.concatenate([edge_weight, jnp.zeros((pad,), jnp.int32)])
    ddip = jnp.concatenate([ddi_weight, jnp.zeros((pad,), jnp.float32)])
    embp = jnp.concatenate([edge_emb[:, 0],
                            jnp.zeros((VOCAB_PAD - VOCAB,), jnp.float32)])

    u, den = _sc_edges(srcp, dstp, ewp, ddip, s16, embp)
    agg = _sc_agg(srcp, dstp.reshape(-1, 128), u, h0quar)
    return _post(agg, den.reshape(NW, N, 4), W_heads, bias_heads)


# speedup vs baseline: 1.0150x; 1.0005x over previous
import functools

import jax
import jax.numpy as jnp
from jax import lax
from jax.experimental import pallas as pl
from jax.experimental.pallas import tpu as pltpu
from jax.experimental.pallas import tpu_sc as plsc

N = 10000
D = 128
H = 3
E = 320000
E_PAD = 327680
VOCAB = 22754
VOCAB_PAD = 22784
DEN_PAD = 40064
NEG_SLOPE = 0.2
BLK = 1000
_HIGH = lax.Precision.HIGHEST

NC = 2
NS = 16
NW = NC * NS

K1_PER_TILE = E_PAD // NW
K1_CHUNK = 512
K1_STEPS = K1_PER_TILE // K1_CHUNK
K2_PER_TILE = E_PAD // NS
K2_CHUNK = 256
K2_STEPS = K2_PER_TILE // K2_CHUNK
NROW = N // NS


def _pre_body(x_ref, wl_ref, bl_ref, wh_ref, asrc_ref, adst_ref,
              h0_ref, s_ref):
    xb = x_ref[...]
    h0 = jnp.dot(xb, wl_ref[...], preferred_element_type=jnp.float32,
                 precision=_HIGH) + bl_ref[...]
    wh = wh_ref[...]
    csrc = lax.dot_general(wh, asrc_ref[...], (((2,), (1,)), ((0,), (0,))),
                           preferred_element_type=jnp.float32,
                           precision=_HIGH)
    cdst = lax.dot_general(wh, adst_ref[...], (((2,), (1,)), ((0,), (0,))),
                           preferred_element_type=jnp.float32,
                           precision=_HIGH)
    z1 = jnp.zeros((1, D), jnp.float32)
    z8 = jnp.zeros((8, D), jnp.float32)
    c16 = jnp.concatenate([csrc, z1, cdst, z1, z8], axis=0)
    s_ref[...] = lax.dot_general(h0, c16, (((1,), (1,)), ((), ())),
                                 preferred_element_type=jnp.float32,
                                 precision=_HIGH)
    for q in range(4):
        h0_ref[q] = h0[:, q * 32:(q + 1) * 32]


def _pre(x, w_lin, b_lin, w_heads, att_src, att_dst):
    return pl.pallas_call(
        _pre_body,
        grid=(N // BLK,),
        in_specs=[
            pl.BlockSpec((BLK, D), lambda i: (i, 0)),
            pl.BlockSpec((D, D), lambda i: (0, 0)),
            pl.BlockSpec((1, D), lambda i: (0, 0)),
            pl.BlockSpec((H, D, D), lambda i: (0, 0, 0)),
            pl.BlockSpec((H, D), lambda i: (0, 0)),
            pl.BlockSpec((H, D), lambda i: (0, 0)),
        ],
        out_specs=[
            pl.BlockSpec((4, BLK, 32), lambda i: (0, i, 0)),
            pl.BlockSpec((BLK, 16), lambda i: (i, 0)),
        ],
        out_shape=[
            jax.ShapeDtypeStruct((4, N, 32), jnp.float32),
            jax.ShapeDtypeStruct((N, 16), jnp.float32),
        ],
    )(x, w_lin, b_lin.reshape(1, D), w_heads, att_src, att_dst)


def _sc_edges_body(src_hbm, dst_hbm, ew_hbm, ddi_hbm, s16_hbm, emb_hbm,
                   u_hbm, den_hbm,
                   srcb, dstb, ewb, ddib, srow, drow, u0b, u1b, u2b,
                   denbuf, embbuf, insem, gsem, usem):
    cid = lax.axis_index("c")
    sid = lax.axis_index("s")
    wid = sid * NC + cid
    iota = lax.iota(jnp.int32, 16)
    ubufs = (u0b, u1b, u2b)

    pltpu.sync_copy(emb_hbm, embbuf)

    def zero_body(i, carry):
        denbuf[pl.ds(i * 16, 16)] = jnp.zeros((16,), jnp.float32)
        return carry
    lax.fori_loop(0, DEN_PAD // 16, zero_body, 0)

    tile_base = wid * K1_PER_TILE

    def in_descs(k, par):
        kc = lax.min(k, K1_STEPS - 1)
        base = tile_base + kc * K1_CHUNK
        return (pltpu.make_async_copy(src_hbm.at[pl.ds(base, K1_CHUNK)],
                                      srcb.at[par], insem),
                pltpu.make_async_copy(dst_hbm.at[pl.ds(base, K1_CHUNK)],
                                      dstb.at[par], insem),
                pltpu.make_async_copy(ew_hbm.at[pl.ds(base, K1_CHUNK)],
                                      ewb.at[par], insem),
                pltpu.make_async_copy(ddi_hbm.at[pl.ds(base, K1_CHUNK)],
                                      ddib.at[par], insem))

    def issue_in(k, par):
        for d in in_descs(k, par):
            d.start()

    def wait_in(k, par):
        for d in in_descs(k, par):
            d.wait()

    def gather_descs(par):
        out = []
        for r in range(K1_CHUNK // 128):
            out.append(pltpu.make_async_copy(
                s16_hbm.at[srcb.at[par, pl.ds(r * 128, 128)]],
                srow.at[par, pl.ds(r * 128, 128)], gsem))
            out.append(pltpu.make_async_copy(
                s16_hbm.at[dstb.at[par, pl.ds(r * 128, 128)]],
                drow.at[par, pl.ds(r * 128, 128)], gsem))
        return out

    def uout_descs(k, par):
        kc = lax.min(k, K1_STEPS - 1)
        base = tile_base + kc * K1_CHUNK
        return tuple(
            pltpu.make_async_copy(ubufs[hd].at[par],
                                  u_hbm.at[hd, pl.ds(base, K1_CHUNK)], usem)
            for hd in range(H))

    for d in in_descs(0, 0):
        d.start()
    for d in in_descs(0, 0):
        d.wait()
    for d in gather_descs(0):
        d.start()

    def chunk_body(k, carry):
        b = lax.rem(k, 2)
        nb = 1 - b
        base = tile_base + k * K1_CHUNK
        issue_in(k + 1, nb)
        for d in gather_descs(b):
            d.wait()

        @pl.when(k != 0)
        def _():
            for d in uout_descs(k - 1, nb):
                d.wait()

        bv = jnp.full((16,), b, jnp.int32)
        for j in range(K1_CHUNK // 16):
            dst_v = dstb[b, pl.ds(j * 16, 16)]
            ewi_v = ewb[b, pl.ds(j * 16, 16)]
            ddi_v = ddib[b, pl.ds(j * 16, 16)]
            ew_v = plsc.load_gather(embbuf, [ewi_v]) - ddi_v
            rowv = iota + (j * 16)
            valid = (iota + (base + j * 16)) < E
            den_base = dst_v * 4
            for hd in range(H):
                s1 = plsc.load_gather(srow, [bv, rowv,
                                             jnp.full((16,), hd, jnp.int32)])
                s2 = plsc.load_gather(drow, [bv, rowv,
                                             jnp.full((16,), hd + 4,
                                                      jnp.int32)])
                a = s1 + s2
                p = jnp.exp(jnp.maximum(a, NEG_SLOPE * a))
                p = jnp.where(valid, p, 0.0)
                plsc.addupdate_scatter(denbuf, [den_base + hd], p)
                ubufs[hd][b, pl.ds(j * 16, 16)] = ew_v * p
        for d in uout_descs(k, b):
            d.start()
        wait_in(k + 1, nb)
        for d in gather_descs(nb):
            d.start()
        return carry

    lax.fori_loop(0, K1_STEPS, chunk_body, 0)
    for d in uout_descs(K1_STEPS - 1, lax.rem(K1_STEPS - 1, 2)):
        d.wait()
    for d in gather_descs(lax.rem(K1_STEPS, 2)):
        d.wait()
    pltpu.sync_copy(denbuf.at[pl.ds(0, N * 4)], den_hbm.at[wid])


def _sc_edges(srcp, dstp, ewp, ddip, s16, emb):
    mesh = plsc.VectorSubcoreMesh(core_axis_name="c", subcore_axis_name="s")
    f = pl.kernel(
        _sc_edges_body,
        out_type=[
            jax.ShapeDtypeStruct((H, E_PAD), jnp.float32),
            jax.ShapeDtypeStruct((NW, N * 4), jnp.float32),
        ],
        mesh=mesh,
        compiler_params=pltpu.CompilerParams(needs_layout_passes=False,
                                             use_tc_tiling_on_sc=False),
        scratch_types=[
            pltpu.VMEM((2, K1_CHUNK), jnp.int32),
            pltpu.VMEM((2, K1_CHUNK), jnp.int32),
            pltpu.VMEM((2, K1_CHUNK), jnp.int32),
            pltpu.VMEM((2, K1_CHUNK), jnp.float32),
            pltpu.VMEM((2, K1_CHUNK, 16), jnp.float32),
            pltpu.VMEM((2, K1_CHUNK, 16), jnp.float32),
            pltpu.VMEM((2, K1_CHUNK), jnp.float32),
            pltpu.VMEM((2, K1_CHUNK), jnp.float32),
            pltpu.VMEM((2, K1_CHUNK), jnp.float32),
            pltpu.VMEM((DEN_PAD,), jnp.float32),
            pltpu.VMEM((VOCAB_PAD,), jnp.float32),
            pltpu.SemaphoreType.DMA,
            pltpu.SemaphoreType.DMA,
            pltpu.SemaphoreType.DMA,
        ],
    )
    return f(srcp, dstp, ewp, ddip, s16, emb)


def _sc_agg_body(src_hbm, dst2_hbm, u_hbm, h0q_hbm, agg_hbm,
                 srcb, dstb2, idxb, ubuf, rbuf, w96,
                 zbuf, acc96, insem, gsem, ssem):
    cid = lax.axis_index("c")
    sid = lax.axis_index("s")
    NG = K2_CHUNK // 128

    def zero_body16(i, carry):
        r = i // 6
        g = i % 6
        zbuf[r, pl.ds(g * 16, 16)] = jnp.zeros((16,), jnp.float32)
        return carry
    lax.fori_loop(0, 125 * 6, zero_body16, 0)

    tile_base = sid * K2_PER_TILE

    def issue_in(k, par):
        kc = lax.min(k, K2_STEPS - 1)
        p3 = lax.rem(k, 3)
        base = tile_base + kc * K2_CHUNK
        row0 = base // 128
        d1 = pltpu.async_copy(src_hbm.at[pl.ds(base, K2_CHUNK)],
                              srcb.at[par], insem)
        d2 = pltpu.async_copy(dst2_hbm.at[pl.ds(row0, NG)],
                              dstb2.at[p3], insem)
        d3 = pltpu.async_copy(
            u_hbm.at[0, pl.ds(base, K2_CHUNK)],
            ubuf.at[pl.ds(par * 3 * K2_CHUNK, K2_CHUNK)], insem)
        d4 = pltpu.async_copy(
            u_hbm.at[1, pl.ds(base, K2_CHUNK)],
            ubuf.at[pl.ds(par * 3 * K2_CHUNK + K2_CHUNK, K2_CHUNK)], insem)
        d5 = pltpu.async_copy(
            u_hbm.at[2, pl.ds(base, K2_CHUNK)],
            ubuf.at[pl.ds(par * 3 * K2_CHUNK + 2 * K2_CHUNK, K2_CHUNK)],
            insem)
        return d1, d2, d3, d4, d5

    def wait_in(k, par):
        for d in issue_wait_in_descs(k, par):
            d.wait()

    def issue_wait_in_descs(k, par):
        kc = lax.min(k, K2_STEPS - 1)
        p3 = lax.rem(k, 3)
        base = tile_base + kc * K2_CHUNK
        row0 = base // 128
        return (pltpu.make_async_copy(src_hbm.at[pl.ds(base, K2_CHUNK)],
                                      srcb.at[par], insem),
                pltpu.make_async_copy(dst2_hbm.at[pl.ds(row0, NG)],
                                      dstb2.at[p3], insem),
                pltpu.make_async_copy(
                    u_hbm.at[0, pl.ds(base, K2_CHUNK)],
                    ubuf.at[pl.ds(par * 3 * K2_CHUNK, K2_CHUNK)], insem),
                pltpu.make_async_copy(
                    u_hbm.at[1, pl.ds(base, K2_CHUNK)],
                    ubuf.at[pl.ds(par * 3 * K2_CHUNK + K2_CHUNK, K2_CHUNK)],
                    insem),
                pltpu.make_async_copy(
                    u_hbm.at[2, pl.ds(base, K2_CHUNK)],
                    ubuf.at[pl.ds(par * 3 * K2_CHUNK + 2 * K2_CHUNK,
                                  K2_CHUNK)], insem))

    def compute_idx_issue_gather(par, qN):
        for g in range(K2_CHUNK // 16):
            r = g // 8
            t0 = (g % 8) * 16
            idxb[par, r, pl.ds(t0, 16)] = srcb[par, pl.ds(g * 16, 16)] + qN
        for r in range(NG):
            pltpu.async_copy(h0q_hbm.at[idxb.at[par, r]],
                             rbuf.at[par, pl.ds(r * 128, 128)], gsem)

    def wait_gather(par):
        for r in range(NG):
            pltpu.make_async_copy(
                h0q_hbm.at[idxb.at[par, r]],
                rbuf.at[par, pl.ds(r * 128, 128)], gsem).wait()

    def wait_scatter_all():
        for _ in range(NG):
            pltpu.make_async_copy(agg_hbm.at[0, pl.ds(0, 128)],
                                  w96.at[pl.ds(0, 128)], ssem).wait()

    for q in range(2):
        qq = cid * 2 + q
        qN = jnp.full((16,), qq * N, jnp.int32)

        for r5 in range(5):
            pltpu.sync_copy(zbuf,
                            acc96.at[pl.ds(sid * NROW + r5 * 125, 125)])
        plsc.subcore_barrier()

        for d in issue_in(0, 0):
            d.wait()
        compute_idx_issue_gather(0, qN)

        def chunk_body(k, carry):
            b = lax.rem(k, 2)
            nb = 1 - b
            issue_in(k + 1, nb)
            wait_gather(b)

            @pl.when(k != 0)
            def _():
                wait_scatter_all()

            ub0 = b * (3 * K2_CHUNK)

            def edge_body(e, carry2):
                i0 = jnp.full((16,), ub0 + e, jnp.int32)
                u0 = plsc.load_gather(ubuf, [i0])
                u1 = plsc.load_gather(ubuf, [i0 + K2_CHUNK])
                u2 = plsc.load_gather(ubuf, [i0 + 2 * K2_CHUNK])
                for g4 in range(2):
                    r_v = rbuf[b, e, pl.ds(g4 * 16, 16)]
                    w96[e, pl.ds(g4 * 16, 16)] = r_v * u0
                    w96[e, pl.ds(32 + g4 * 16, 16)] = r_v * u1
                    w96[e, pl.ds(64 + g4 * 16, 16)] = r_v * u2
                return carry2

            p3 = lax.rem(k, 3)
            for r in range(NG):
                lax.fori_loop(r * 128, (r + 1) * 128, edge_body, 0,
                              unroll=8)
                pltpu.async_copy(w96.at[pl.ds(r * 128, 128)],
                                 acc96.at[dstb2.at[p3, r]], ssem, add=True)

            wait_in(k + 1, nb)
            compute_idx_issue_gather(nb, qN)
            return carry

        lax.fori_loop(0, K2_STEPS, chunk_body, 0)
        wait_scatter_all()
        wait_gather(lax.rem(K2_STEPS, 2))
        plsc.subcore_barrier()
        pltpu.sync_copy(acc96.at[pl.ds(sid * NROW, NROW)],
                        agg_hbm.at[qq, pl.ds(sid * NROW, NROW)])
        plsc.subcore_barrier()


def _sc_agg(srcp, dst2, u, h0quar):
    mesh = plsc.VectorSubcoreMesh(core_axis_name="c", subcore_axis_name="s")
    f = pl.kernel(
        _sc_agg_body,
        out_type=jax.ShapeDtypeStruct((4, N, 96), jnp.float32),
        mesh=mesh,
        compiler_params=pltpu.CompilerParams(needs_layout_passes=False,
                                             use_tc_tiling_on_sc=False),
        scratch_types=[
            pltpu.VMEM((2, K2_CHUNK), jnp.int32),
            pltpu.VMEM((3, K2_CHUNK // 128, 128), jnp.int32),
            pltpu.VMEM((2, K2_CHUNK // 128, 128), jnp.int32),
            pltpu.VMEM((2 * H * K2_CHUNK,), jnp.float32),
            pltpu.VMEM((2, K2_CHUNK, 32), jnp.float32),
            pltpu.VMEM((K2_CHUNK, 96), jnp.float32),
            pltpu.VMEM((125, 96), jnp.float32),
            pltpu.VMEM_SHARED((N, 96), jnp.float32),
            pltpu.SemaphoreType.DMA,
            pltpu.SemaphoreType.DMA,
            pltpu.SemaphoreType.DMA,
        ],
    )
    return f(srcp, dst2, u, h0quar)


def _post_body(agg_ref, den_ref, wh_ref, bias_ref, out_ref):
    den = jnp.sum(den_ref[...], axis=0)
    acc = jnp.zeros((BLK, D), jnp.float32)
    for hd in range(H):
        inv = 1.0 / (den[:, hd][:, None] + 1e-16)
        for qq in range(4):
            acc = acc + jnp.dot(agg_ref[qq][:, hd * 32:(hd + 1) * 32] * inv,
                                wh_ref[hd, qq * 32:(qq + 1) * 32, :],
                                preferred_element_type=jnp.float32,
                                precision=_HIGH)
    out_ref[...] = acc * (1.0 / H) + jnp.mean(bias_ref[...], axis=0)


def _post(agg, den32, w_heads, bias_heads):
    return pl.pallas_call(
        _post_body,
        grid=(N // BLK,),
        in_specs=[
            pl.BlockSpec((4, BLK, 96), lambda i: (0, i, 0)),
            pl.BlockSpec((NW, BLK, 4), lambda i: (0, i, 0)),
            pl.BlockSpec((H, D, D), lambda i: (0, 0, 0)),
            pl.BlockSpec((H, D), lambda i: (0, 0)),
        ],
        out_specs=pl.BlockSpec((BLK, D), lambda i: (i, 0)),
        out_shape=jax.ShapeDtypeStruct((N, D), jnp.float32),
    )(agg, den32, w_heads, bias_heads)


def kernel(x, edge_index, edge_weight, ddi_weight, W_lin, b_lin, edge_emb,
           W_heads, att_src, att_dst, bias_heads):
    h0pair, s16 = _pre(x, W_lin, b_lin, W_heads, att_src, att_dst)
    h0quar = h0pair.reshape(4 * N, 32)

    pad = E_PAD - E
    spread = (jnp.arange(pad, dtype=jnp.int32) * 13) % N
    srcp = jnp.concatenate([edge_index[0], spread])
    dstp = jnp.concatenate([edge_index[1], spread])
    ewp = jnp.concatenate([edge_weight, jnp.zeros((pad,), jnp.int32)])
    ddip = jnp.concatenate([ddi_weight, jnp.zeros((pad,), jnp.float32)])
    embp = jnp.concatenate([edge_emb[:, 0],
                            jnp.zeros((VOCAB_PAD - VOCAB,), jnp.float32)])

    u, den = _sc_edges(srcp, dstp, ewp, ddip, s16, embp)
    agg = _sc_agg(srcp, dstp.reshape(-1, 128), u, h0quar)
    return _post(agg, den.reshape(NW, N, 4), W_heads, bias_heads)
